# baseline (speedup 1.0000x reference)
import functools

import jax
import jax.numpy as jnp
from jax.experimental import pallas as pl
from jax.experimental.pallas import tpu as pltpu
from jax.experimental.pallas import tpu_sc as plsc

_N = 32768
_C = 64
_K = 4096
_KR = 4096
_BN = 512


def _byte_planes(table):
    tu = jax.lax.bitcast_convert_type(table, jnp.int32)
    planes = [(((tu >> s) & 0xFF) - 128).astype(jnp.int8)
              for s in (0, 8, 16, 24)]
    return jnp.concatenate(planes, axis=1)


def _exact_gather(planes, idx, k):
    n = idx.shape[0]
    iota = jax.lax.broadcasted_iota(jnp.int32, (n, k), 1)
    onehot = (iota == idx[:, None]).astype(jnp.int8)
    g = jax.lax.dot_general(onehot, planes, (((1,), (0,)), ((), ())),
                            preferred_element_type=jnp.int32) + 128
    acc = g[:, 0:_C]
    acc = acc | (g[:, _C:2 * _C] << 8)
    acc = acc | (g[:, 2 * _C:3 * _C] << 16)
    acc = acc | (g[:, 3 * _C:4 * _C] << 24)
    return jax.lax.bitcast_convert_type(acc, jnp.float32)


_W = 128


def _argmin_scan(x_sq, xc2, csq, k):
    n = x_sq.shape[0]
    best = jnp.full((n, _W), jnp.inf, jnp.float32)
    bests = jnp.zeros((n, _W), jnp.int32)
    for s in range(k // _W):
        d_s = (x_sq + xc2[:, s * _W:(s + 1) * _W]) + csq[:, s * _W:(s + 1) * _W]
        cond = d_s < best
        best = jnp.where(cond, d_s, best)
        bests = jnp.where(cond, s, bests)
    mind = jnp.min(best, axis=1, keepdims=True)
    lane = jax.lax.broadcasted_iota(jnp.int32, (n, _W), 1)
    cand = jnp.where(best == mind, bests * _W + lane, k)
    return jnp.min(cand, axis=1)


def _rvq_body(x_ref, cbm2_ref, rcbm2_ref, cbsq_ref, rcbsq_ref,
              cbpl_ref,
              quant_ref, idx_ref, idxr_ref,
              r_buf):
    i = pl.program_id(0)

    r = r_buf[(i + 1) % 2]
    r_sq = jnp.sum(r * r, axis=1, keepdims=True)
    rc2 = jax.lax.dot_general(r, rcbm2_ref[...], (((1,), (1,)), ((), ())),
                              preferred_element_type=jnp.float32)
    idx2 = _argmin_scan(r_sq, rc2, rcbsq_ref[...], _KR)
    idxr_ref[...] = idx2[:, None]

    x = x_ref[...]
    x_sq = jnp.sum(x * x, axis=1, keepdims=True)
    xc2 = jax.lax.dot_general(x, cbm2_ref[...], (((1,), (1,)), ((), ())),
                              preferred_element_type=jnp.float32)
    idx = _argmin_scan(x_sq, xc2, cbsq_ref[...], _K)
    quant = _exact_gather(cbpl_ref[...], idx, _K)
    quant_ref[...] = quant
    idx_ref[...] = idx[:, None]
    r_buf[i % 2] = x - quant


_NC = 2
_NS = 16
_NW = _NC * _NS
_BPW = _N // _NW


_CP = 128
_CH = 512


def _sc_gather(table_pad, idx):
    mesh = plsc.VectorSubcoreMesh(core_axis_name="c", subcore_axis_name="s")

    @functools.partial(
        pl.kernel, mesh=mesh,
        out_type=jax.ShapeDtypeStruct((_N, _CP), jnp.float32),
        scratch_types=[
            pltpu.VMEM((_CH,), jnp.int32),
            pltpu.VMEM((_CH, _CP), jnp.float32),
            pltpu.SemaphoreType.DMA,
        ],
    )
    def k(table_hbm, idx_hbm, out_hbm, idx_v, rows_v, sem):
        wid = jax.lax.axis_index("s") * _NC + jax.lax.axis_index("c")
        for c in range(_BPW // _CH):
            base = wid * _BPW + c * _CH
            pltpu.sync_copy(idx_hbm.at[pl.ds(base, _CH)], idx_v)
            pltpu.async_copy(table_hbm.at[idx_v], rows_v, sem).wait()
            pltpu.sync_copy(rows_v, out_hbm.at[pl.ds(base, _CH)])

    return k(table_pad, idx)


def kernel(x, codebook, residual_codebook):
    cbm2 = -2.0 * codebook
    rcbm2 = -2.0 * residual_codebook
    cbsq = jnp.sum(codebook * codebook, axis=1)[None, :]
    rcbsq = jnp.sum(residual_codebook * residual_codebook, axis=1)[None, :]
    cbpl = _byte_planes(codebook)

    nb = _N // _BN
    s1 = lambda i: (jnp.minimum(i, nb - 1), 0)
    s2 = lambda i: (jnp.maximum(i - 1, 0), 0)
    const = lambda i: (0, 0)
    out = pl.pallas_call(
        _rvq_body,
        grid=(nb + 1,),
        in_specs=[
            pl.BlockSpec((_BN, _C), s1),
            pl.BlockSpec((_K, _C), const),
            pl.BlockSpec((_KR, _C), const),
            pl.BlockSpec((1, _K), const),
            pl.BlockSpec((1, _KR), const),
            pl.BlockSpec((_K, 4 * _C), const),
        ],
        out_specs=[
            pl.BlockSpec((_BN, _C), s1),
            pl.BlockSpec((_BN, 1), s1),
            pl.BlockSpec((_BN, 1), s2),
        ],
        out_shape=[
            jax.ShapeDtypeStruct((_N, _C), jnp.float32),
            jax.ShapeDtypeStruct((_N, 1), jnp.int32),
            jax.ShapeDtypeStruct((_N, 1), jnp.int32),
        ],
        scratch_shapes=[pltpu.VMEM((2, _BN, _C), jnp.float32)],
    )(x, cbm2, rcbm2, cbsq, rcbsq, cbpl)
    quant, idx, idx_r = out
    idx_r = idx_r[:, 0]
    rcb_pad = jnp.pad(residual_codebook, ((0, 0), (0, _CP - _C)))
    quant_r = _sc_gather(rcb_pad, idx_r)[:, :_C]
    return (quant, idx[:, 0], quant_r, idx_r)
